# Initial kernel scaffold; baseline (speedup 1.0000x reference)
import jax
import jax.numpy as jnp
from jax.experimental import pallas as pl


def kernel(edge_index, edge_attr, num_nodes, W1, b1, W2, b2):
    raise NotImplementedError("write your pallas kernel here")


# speedup vs baseline: 4.3505x; 4.3505x over previous
import functools

import jax
import jax.numpy as jnp
from jax import lax
from jax.experimental import pallas as pl
from jax.experimental.pallas import tpu as pltpu
from jax.experimental.pallas import tpu_sc as plsc

E = 65536
N = 2048
EDGE_DIM = 16
H = 8
KEYS = N * N
NC = 2
NS = 16
NW = NC * NS
RANGE = KEYS // NW
ZROWS = 1024
KCH = 8192
MCAP = 8192
BM = 2048
NBMAX = MCAP // BM
SUBR = 65536


def _mlp_body(ei_ref, x_ref, w1_ref, b1_ref, w2_ref, b2_ref, eb_ref, key_ref):
    h = jnp.maximum(
        jnp.dot(x_ref[...], w1_ref[...], preferred_element_type=jnp.float32)
        + b1_ref[...], 0.0)
    eb_ref[...] = (
        jnp.dot(h, w2_ref[...], preferred_element_type=jnp.float32)
        + b2_ref[...])
    k = ei_ref[0, :] * N + ei_ref[1, :]
    key_ref[...] = k.reshape(key_ref.shape)


_G = 8
_EB = E // _G


_mlp_call = pl.pallas_call(
    _mlp_body,
    grid=(_G,),
    in_specs=[
        pl.BlockSpec((2, _EB), lambda g: (0, g)),
        pl.BlockSpec((_EB, EDGE_DIM), lambda g: (g, 0)),
        pl.BlockSpec((EDGE_DIM, EDGE_DIM), lambda g: (0, 0)),
        pl.BlockSpec((1, EDGE_DIM), lambda g: (0, 0)),
        pl.BlockSpec((EDGE_DIM, H), lambda g: (0, 0)),
        pl.BlockSpec((1, H), lambda g: (0, 0)),
    ],
    out_specs=[
        pl.BlockSpec((_EB, H), lambda g: (g, 0)),
        pl.BlockSpec((_EB // 128, 128), lambda g: (g, 0)),
    ],
    out_shape=[
        jax.ShapeDtypeStruct((E, H), jnp.float32),
        jax.ShapeDtypeStruct((E // 128, 128), jnp.int32),
    ],
)


_SH16 = 11


def _midx(pos):
    sh = jnp.full((16,), _SH16, jnp.int32)
    mskc = jnp.full((16,), BM - 1, jnp.int32)
    return [pos >> sh, pos & mskc]


def _sc_body(keys_hbm, ebias_hbm, zeros_hbm, out_hbm,
             zbuf, kbuf, mkeys, mids, rows, tagv, zsem, gsem, ssem):
    cid = lax.axis_index("c")
    sid = lax.axis_index("s")
    wid = sid * NC + cid
    lo = wid * RANGE
    hi = lo + RANGE

    pltpu.sync_copy(zeros_hbm, zbuf)

    def fire_zero(i, _):
        pltpu.make_async_copy(
            zbuf, out_hbm.at[pl.ds(lo + i * ZROWS, ZROWS)], zsem).start()
        return 0

    lax.fori_loop(0, RANGE // ZROWS, fire_zero, 0)

    def drain_zero(i, _):
        pltpu.make_async_copy(
            zbuf, out_hbm.at[pl.ds(lo + i * ZROWS, ZROWS)], zsem).wait()
        return 0

    iota = lax.broadcasted_iota(jnp.int32, (16,), 0)

    _BISECT = 4

    if _BISECT < 4:
        lax.fori_loop(0, RANGE // ZROWS, drain_zero, 0)
    if _BISECT == 1:
        return

    lo16 = jnp.full((16,), lo, jnp.int32)
    hi16 = jnp.full((16,), hi, jnp.int32)
    one16 = jnp.full((16,), 1, jnp.int32)
    cap16 = jnp.full((16,), MCAP - 1, jnp.int32)

    def scan_chunk(c, m):
        pltpu.sync_copy(keys_hbm.at[pl.ds(c * KCH, KCH)], kbuf)
        cbase = c * KCH

        def scan_vreg(j, m):
            k16 = kbuf[pl.ds(j * 16, 16)]
            msk = (k16 >= lo16) & (k16 < hi16)
            c16 = jnp.cumsum(msk.astype(jnp.int32))
            m16 = jnp.full((16,), m, jnp.int32)
            pos = jnp.minimum(m16 + c16 - one16, cap16)
            e16 = jnp.full((16,), cbase + j * 16, jnp.int32) + iota
            plsc.store_scatter(mkeys, _midx(pos), k16, mask=msk)
            plsc.store_scatter(mids, _midx(pos), e16, mask=msk)
            return jnp.minimum(m + jnp.max(c16), MCAP)

        return lax.fori_loop(0, KCH // 16, scan_vreg, m)

    m = lax.fori_loop(0, E // KCH, scan_chunk, jnp.int32(0))

    if _BISECT == 2:
        return

    mm116 = jnp.full((16,), jnp.maximum(m - 1, 0), jnp.int32)
    zero16 = jnp.full((16,), 0, jnp.int32)
    sr16 = jnp.full((16,), SUBR, jnp.int32)
    nv = (m + 15) >> 4

    for s in range(RANGE // SUBR):
        sublo16 = jnp.full((16,), lo + s * SUBR, jnp.int32)
        subhi16 = sublo16 + sr16

        def tag_store(v, _):
            pos16 = jnp.full((16,), v * 16, jnp.int32) + iota
            valid = pos16 <= mm116
            pos_c = jnp.minimum(pos16, mm116)
            kk = plsc.load_gather(mkeys, _midx(pos_c))
            insub = valid & (kk >= sublo16) & (kk < subhi16)
            kidx = jnp.minimum(jnp.maximum(kk - sublo16, zero16),
                               sr16 - one16)
            plsc.store_scatter(tagv, [kidx], pos16, mask=insub)
            return 0

        def id_rewrite(v, _):
            pos16 = jnp.full((16,), v * 16, jnp.int32) + iota
            valid = pos16 <= mm116
            pos_c = jnp.minimum(pos16, mm116)
            kk = plsc.load_gather(mkeys, _midx(pos_c))
            insub = valid & (kk >= sublo16) & (kk < subhi16)
            kidx = jnp.minimum(jnp.maximum(kk - sublo16, zero16),
                               sr16 - one16)
            w16 = plsc.load_gather(tagv, [kidx])
            w_c = jnp.minimum(jnp.maximum(w16, zero16), mm116)
            wid16 = plsc.load_gather(mids, _midx(w_c))
            plsc.store_scatter(mids, _midx(pos_c), wid16, mask=insub)
            return 0

        lax.fori_loop(0, nv, tag_store, 0)
        lax.fori_loop(0, nv, id_rewrite, 0)

    mm1 = jnp.maximum(m - 1, 0)
    key_last = plsc.load_gather(mkeys, _midx(jnp.full((16,), mm1, jnp.int32)))
    id_last = plsc.load_gather(mids, _midx(jnp.full((16,), mm1, jnp.int32)))
    nb = (m + BM - 1) >> 11

    def pad_slot(j, _):
        posv = j * 16 + iota
        pmsk = posv >= m
        plsc.store_scatter(mkeys, _midx(posv), key_last, mask=pmsk)
        plsc.store_scatter(mids, _midx(posv), id_last, mask=pmsk)
        return 0

    lax.fori_loop(m >> 4, nb << 7, pad_slot, 0)

    if _BISECT == 3:
        return

    lax.fori_loop(0, RANGE // ZROWS, drain_zero, 0)

    def do_batch(b, _):
        g = pltpu.make_async_copy(ebias_hbm.at[mids.at[b]], rows, gsem)
        g.start()
        g.wait()
        s = pltpu.make_async_copy(rows, out_hbm.at[mkeys.at[b]], ssem)
        s.start()
        s.wait()
        return 0

    lax.fori_loop(0, nb, do_batch, 0)


_sc_call = functools.partial(
    pl.kernel,
    out_type=jax.ShapeDtypeStruct((KEYS, H), jnp.float32),
    mesh=plsc.VectorSubcoreMesh(core_axis_name="c", subcore_axis_name="s"),
    compiler_params=pltpu.CompilerParams(
        needs_layout_passes=False, use_tc_tiling_on_sc=False),
    scratch_types=[
        pltpu.VMEM((ZROWS, H), jnp.float32),
        pltpu.VMEM((KCH,), jnp.int32),
        pltpu.VMEM((NBMAX, BM), jnp.int32),
        pltpu.VMEM((NBMAX, BM), jnp.int32),
        pltpu.VMEM((BM, H), jnp.float32),
        pltpu.VMEM((SUBR,), jnp.int32),
        pltpu.SemaphoreType.DMA,
        pltpu.SemaphoreType.DMA,
        pltpu.SemaphoreType.DMA,
    ],
)(_sc_body)


def kernel(edge_index, edge_attr, num_nodes, W1, b1, W2, b2):
    ebias, keys2d = _mlp_call(
        edge_index, edge_attr, W1, b1.reshape(1, EDGE_DIM),
        W2, b2.reshape(1, H))
    keys = keys2d.reshape(E)
    zeros_in = jnp.zeros((ZROWS, H), jnp.float32)
    out = _sc_call(keys, ebias, zeros_in)
    return out.reshape(N, N, H)
